# initial kernel scaffold (unmeasured)
import jax
import jax.numpy as jnp
from jax import lax
from jax.experimental import pallas as pl
from jax.experimental.pallas import tpu as pltpu


def kernel(
    x,
):
    def body(*refs):
        pass

    out_shape = jax.ShapeDtypeStruct(..., jnp.float32)
    return pl.pallas_call(body, out_shape=out_shape)(...)


# baseline (device time: 788628 ns/iter reference)
import jax
import jax.numpy as jnp
from jax import lax
from jax.experimental import pallas as pl
from jax.experimental.pallas import tpu as pltpu

HALF = 8192


def _comm_body(x_ref, recv_ref, send_sem_x, recv_sem_x, send_sem_y, recv_sem_y):
    my_x = lax.axis_index("x")
    my_y = lax.axis_index("y")
    x_nbr = (1 - my_x, my_y)
    y_nbr = (my_x, 1 - my_y)

    barrier = pltpu.get_barrier_semaphore()
    for nbr in (x_nbr, y_nbr):
        pl.semaphore_signal(
            barrier, inc=1, device_id=nbr, device_id_type=pl.DeviceIdType.MESH
        )
    pl.semaphore_wait(barrier, 2)

    rows = pl.ds(my_y * HALF, HALF)

    rdma_x = pltpu.make_async_remote_copy(
        src_ref=x_ref.at[rows, :],
        dst_ref=recv_ref.at[rows, :],
        send_sem=send_sem_x,
        recv_sem=recv_sem_x,
        device_id=x_nbr,
        device_id_type=pl.DeviceIdType.MESH,
    )
    rdma_x.start()
    rdma_x.wait()

    rdma_y = pltpu.make_async_remote_copy(
        src_ref=recv_ref.at[rows, :],
        dst_ref=recv_ref.at[rows, :],
        send_sem=send_sem_y,
        recv_sem=recv_sem_y,
        device_id=y_nbr,
        device_id_type=pl.DeviceIdType.MESH,
    )
    rdma_y.start()
    rdma_y.wait()


def _add_body(x_ref, r_ref, o_ref):
    o_ref[...] = x_ref[...] + r_ref[...]


def kernel(x):
    m, n = x.shape
    recv = pl.pallas_call(
        _comm_body,
        out_shape=jax.ShapeDtypeStruct((m, n), x.dtype),
        in_specs=[pl.BlockSpec(memory_space=pl.ANY)],
        out_specs=pl.BlockSpec(memory_space=pl.ANY),
        scratch_shapes=[pltpu.SemaphoreType.DMA] * 4,
        compiler_params=pltpu.CompilerParams(collective_id=0),
    )(x)

    blk = 1024
    out = pl.pallas_call(
        _add_body,
        out_shape=jax.ShapeDtypeStruct((m, n), x.dtype),
        grid=(m // blk,),
        in_specs=[
            pl.BlockSpec((blk, n), lambda i: (i, 0)),
            pl.BlockSpec((blk, n), lambda i: (i, 0)),
        ],
        out_specs=pl.BlockSpec((blk, n), lambda i: (i, 0)),
    )(x, recv)
    return out


# device time: 452309 ns/iter; 1.7436x vs baseline; 1.7436x over previous
import jax
import jax.numpy as jnp
from jax import lax
from jax.experimental import pallas as pl
from jax.experimental.pallas import tpu as pltpu

HALF = 8192
K = 16
RCH = HALF // K


def _comm_body(x_ref, recv_ref, send_sem_x, recv_sem_x, send_sem_y, recv_sem_y):
    my_x = lax.axis_index("x")
    my_y = lax.axis_index("y")
    x_nbr = (1 - my_x, my_y)
    y_nbr = (my_x, 1 - my_y)

    barrier = pltpu.get_barrier_semaphore()
    for nbr in (x_nbr, y_nbr):
        pl.semaphore_signal(
            barrier, inc=1, device_id=nbr, device_id_type=pl.DeviceIdType.MESH
        )
    pl.semaphore_wait(barrier, 2)

    base = my_y * HALF

    def rows(c):
        return pl.ds(base + c * RCH, RCH)

    rdma_x = []
    for c in range(K):
        r = pltpu.make_async_remote_copy(
            src_ref=x_ref.at[rows(c), :],
            dst_ref=recv_ref.at[rows(c), :],
            send_sem=send_sem_x.at[c],
            recv_sem=recv_sem_x.at[c],
            device_id=x_nbr,
            device_id_type=pl.DeviceIdType.MESH,
        )
        r.start()
        rdma_x.append(r)

    rdma_y = []
    for c in range(K):
        rdma_x[c].wait_recv()
        r = pltpu.make_async_remote_copy(
            src_ref=recv_ref.at[rows(c), :],
            dst_ref=recv_ref.at[rows(c), :],
            send_sem=send_sem_y.at[c],
            recv_sem=recv_sem_y.at[c],
            device_id=y_nbr,
            device_id_type=pl.DeviceIdType.MESH,
        )
        r.start()
        rdma_y.append(r)

    for c in range(K):
        rdma_x[c].wait_send()
        rdma_y[c].wait()


def _add_body(x_ref, r_ref, o_ref):
    o_ref[...] = x_ref[...] + r_ref[...]


def kernel(x):
    m, n = x.shape
    recv = pl.pallas_call(
        _comm_body,
        out_shape=jax.ShapeDtypeStruct((m, n), x.dtype),
        in_specs=[pl.BlockSpec(memory_space=pl.ANY)],
        out_specs=pl.BlockSpec(memory_space=pl.ANY),
        scratch_shapes=[pltpu.SemaphoreType.DMA((K,))] * 4,
        compiler_params=pltpu.CompilerParams(collective_id=0),
    )(x)

    blk = 1024
    out = pl.pallas_call(
        _add_body,
        out_shape=jax.ShapeDtypeStruct((m, n), x.dtype),
        grid=(m // blk,),
        in_specs=[
            pl.BlockSpec((blk, n), lambda i: (i, 0)),
            pl.BlockSpec((blk, n), lambda i: (i, 0)),
        ],
        out_specs=pl.BlockSpec((blk, n), lambda i: (i, 0)),
    )(x, recv)
    return out
